# baseline (speedup 1.0000x reference)
import functools

import jax
import jax.numpy as jnp
from jax import lax
from jax.experimental import pallas as pl
from jax.experimental.pallas import tpu as pltpu
from jax.experimental.pallas import tpu_sc as plsc

B = 16
S = 2048
T = 16384
D = 512

NW = 32
CH = 32
NPOS = S // CH
NSLOT = 32
NBUF = 6
LAG = 3

_mesh = plsc.VectorSubcoreMesh(core_axis_name="c", subcore_axis_name="s")


@functools.partial(
    pl.kernel,
    mesh=_mesh,
    out_type=jax.ShapeDtypeStruct((B, S, D), jnp.float32),
    scratch_types=[
        pltpu.VMEM((32,), jnp.int32),
        pltpu.VMEM((NBUF, CH), jnp.int32),
        pltpu.VMEM((CH, D), jnp.float32),
        pltpu.VMEM((CH, D), jnp.float32),
        pltpu.VMEM((CH, D), jnp.float32),
        pltpu.VMEM((CH, D), jnp.float32),
        pltpu.VMEM((CH, D), jnp.float32),
        pltpu.VMEM((CH, D), jnp.float32),
        pltpu.VMEM((CH, D), jnp.float32),
        pltpu.SemaphoreType.DMA,
        pltpu.SemaphoreType.DMA,
        pltpu.SemaphoreType.DMA,
        pltpu.SemaphoreType.DMA,
        pltpu.SemaphoreType.DMA,
        pltpu.SemaphoreType.DMA,
        pltpu.SemaphoreType.DMA,
        pltpu.SemaphoreType.DMA,
        pltpu.SemaphoreType.DMA,
        pltpu.SemaphoreType.DMA,
        pltpu.SemaphoreType.DMA,
        pltpu.SemaphoreType.DMA,
        pltpu.SemaphoreType.DMA,
    ],
)
def _sc_unflatten(
    flat_hbm, cu_hbm, out_hbm, cu_v, idx_v,
    buf0, buf1, buf2, buf3, buf4, buf5, zbuf,
    sg0, sg1, sg2, sg3, sg4, sg5,
    sw0, sw1, sw2, sw3, sw4, sw5, sem_z,
):
    buf = [buf0, buf1, buf2, buf3, buf4, buf5]
    sem_g = [sg0, sg1, sg2, sg3, sg4, sg5]
    sem_w = [sw0, sw1, sw2, sw3, sw4, sw5]
    wid = lax.axis_index("c") * 16 + lax.axis_index("s")

    cu_cp = pltpu.async_copy(cu_hbm, cu_v, sg0)

    def _zero(i, carry):
        r = i // 4
        col = (i % 4) * 128
        for u in range(8):
            zbuf[r, pl.ds(col + u * 16, 16)] = jnp.zeros((16,), jnp.float32)
        return carry

    lax.fori_loop(0, CH * D // 128, _zero, 0)
    cu_cp.wait()

    lane = lax.broadcasted_iota(jnp.int32, (16,), 0)

    start_e = []
    end_e = []
    for t in range(B):
        cu_b = cu_v[pl.ds(t, 16)]
        start_e.append(cu_b[0])
        end_e.append(cu_b[1])

    pos = []
    nv_s = []
    base_s = []
    end_s = []
    valid = []
    for s in range(NSLOT):
        t = s % B
        p = (wid + 2 * t + CH * (s // B)) % NPOS
        nv = jnp.clip(end_e[t] - start_e[t] - p * CH, 0, CH)
        pos.append(p)
        nv_s.append(nv)
        base_s.append(start_e[t] + p * CH)
        end_s.append(end_e[t])
        valid.append(nv > 0)

    def _dst(s):
        return out_hbm.at[s % B, pl.ds(pos[s] * CH, CH)]

    for s in range(NSLOT):
        @pl.when(jnp.logical_not(valid[s]))
        def _():
            pltpu.async_copy(zbuf, _dst(s), sem_z)

    for s in range(NSLOT + LAG):
        if s < NSLOT:
            k = s % NBUF
            if s >= NBUF:
                @pl.when(valid[s - NBUF])
                def _():
                    pltpu.make_async_copy(
                        buf[k], _dst(s - NBUF), sem_w[k]
                    ).wait()

            @pl.when(valid[s])
            def _():
                for j in range(CH // 16):
                    idx_v[k, pl.ds(j * 16, 16)] = jnp.minimum(
                        base_s[s] + j * 16 + lane, end_s[s] - 1
                    )
                pltpu.async_copy(
                    flat_hbm.at[idx_v.at[k]], buf[k], sem_g[k]
                )

        sp = s - LAG
        if sp >= 0:
            kp = sp % NBUF

            @pl.when(valid[sp])
            def _():
                pltpu.make_async_copy(buf[kp], _dst(sp), sem_g[kp]).wait()
                nv = nv_s[sp]
                def _ztail(i, carry):
                    r = nv + i // 4
                    col = (i % 4) * 128
                    for u in range(8):
                        buf[kp][r, pl.ds(col + u * 16, 16)] = jnp.zeros(
                            (16,), jnp.float32
                        )
                    return carry

                lax.fori_loop(0, (CH - nv) * (D // 128), _ztail, 0)
                pltpu.async_copy(buf[kp], _dst(sp), sem_w[kp])

    for s in range(NSLOT - NBUF, NSLOT):
        @pl.when(valid[s])
        def _():
            pltpu.make_async_copy(
                buf[s % NBUF], _dst(s), sem_w[s % NBUF]
            ).wait()

    n_zero = 0
    for s in range(NSLOT):
        n_zero = n_zero + jnp.where(valid[s], 0, 1)

    def _drain_z(i, carry):
        pltpu.make_async_copy(
            zbuf, out_hbm.at[0, pl.ds(0, CH)], sem_z
        ).wait()
        return carry

    lax.fori_loop(0, n_zero, _drain_z, 0)


def _mask_body(cu_ref, mask_ref):
    col = lax.broadcasted_iota(jnp.int32, (1, S), 1)
    for b in range(B):
        ln = cu_ref[b + 1] - cu_ref[b]
        mask_ref[pl.ds(b, 1), :] = col < ln


_mask_call = pl.pallas_call(
    _mask_body,
    in_specs=[pl.BlockSpec(memory_space=pltpu.SMEM)],
    out_specs=pl.BlockSpec(memory_space=pltpu.VMEM),
    out_shape=jax.ShapeDtypeStruct((B, S), jnp.bool_),
)


def kernel(flat, cu_seqlens):
    cu_p = jnp.pad(cu_seqlens.astype(jnp.int32), (0, 32 - (B + 1)))
    out = _sc_unflatten(flat, cu_p)
    mask = _mask_call(cu_p)
    return out, mask


# speedup vs baseline: 1.0079x; 1.0079x over previous
import functools

import jax
import jax.numpy as jnp
from jax import lax
from jax.experimental import pallas as pl
from jax.experimental.pallas import tpu as pltpu
from jax.experimental.pallas import tpu_sc as plsc

B = 16
S = 2048
T = 16384
D = 512

NW = 32
CH = 32
NPOS = S // CH
NSLOT = 32
NBUF = 7
ZR = 16
LAG = 3

_mesh = plsc.VectorSubcoreMesh(core_axis_name="c", subcore_axis_name="s")


@functools.partial(
    pl.kernel,
    mesh=_mesh,
    out_type=jax.ShapeDtypeStruct((B, S, D), jnp.float32),
    scratch_types=[
        pltpu.VMEM((32,), jnp.int32),
        pltpu.VMEM((NBUF, CH), jnp.int32),
        pltpu.VMEM((CH, D), jnp.float32),
        pltpu.VMEM((CH, D), jnp.float32),
        pltpu.VMEM((CH, D), jnp.float32),
        pltpu.VMEM((CH, D), jnp.float32),
        pltpu.VMEM((CH, D), jnp.float32),
        pltpu.VMEM((CH, D), jnp.float32),
        pltpu.VMEM((CH, D), jnp.float32),
        pltpu.VMEM((ZR, D), jnp.float32),
        pltpu.SemaphoreType.DMA,
        pltpu.SemaphoreType.DMA,
        pltpu.SemaphoreType.DMA,
        pltpu.SemaphoreType.DMA,
        pltpu.SemaphoreType.DMA,
        pltpu.SemaphoreType.DMA,
        pltpu.SemaphoreType.DMA,
        pltpu.SemaphoreType.DMA,
        pltpu.SemaphoreType.DMA,
        pltpu.SemaphoreType.DMA,
        pltpu.SemaphoreType.DMA,
        pltpu.SemaphoreType.DMA,
        pltpu.SemaphoreType.DMA,
        pltpu.SemaphoreType.DMA,
        pltpu.SemaphoreType.DMA,
    ],
)
def _sc_unflatten(
    flat_hbm, cu_hbm, out_hbm, cu_v, idx_v,
    buf0, buf1, buf2, buf3, buf4, buf5, buf6, zbuf,
    sg0, sg1, sg2, sg3, sg4, sg5, sg6,
    sw0, sw1, sw2, sw3, sw4, sw5, sw6, sem_z,
):
    buf = [buf0, buf1, buf2, buf3, buf4, buf5, buf6]
    sem_g = [sg0, sg1, sg2, sg3, sg4, sg5, sg6]
    sem_w = [sw0, sw1, sw2, sw3, sw4, sw5, sw6]
    wid = lax.axis_index("c") * 16 + lax.axis_index("s")

    cu_cp = pltpu.async_copy(cu_hbm, cu_v, sg0)

    def _zero(i, carry):
        r = i // 4
        col = (i % 4) * 128
        for u in range(8):
            zbuf[r, pl.ds(col + u * 16, 16)] = jnp.zeros((16,), jnp.float32)
        return carry

    lax.fori_loop(0, ZR * D // 128, _zero, 0)
    cu_cp.wait()

    lane = lax.broadcasted_iota(jnp.int32, (16,), 0)

    start_e = []
    end_e = []
    for t in range(B):
        cu_b = cu_v[pl.ds(t, 16)]
        start_e.append(cu_b[0])
        end_e.append(cu_b[1])

    pos = []
    nv_s = []
    base_s = []
    end_s = []
    valid = []
    for s in range(NSLOT):
        t = s % B
        p = (wid + 2 * t + CH * (s // B)) % NPOS
        nv = jnp.clip(end_e[t] - start_e[t] - p * CH, 0, CH)
        pos.append(p)
        nv_s.append(nv)
        base_s.append(start_e[t] + p * CH)
        end_s.append(end_e[t])
        valid.append(nv > 0)

    def _dst(s):
        return out_hbm.at[s % B, pl.ds(pos[s] * CH, CH)]

    for s in range(NSLOT + LAG):
        if s < NSLOT:
            k = s % NBUF
            if s >= NBUF:
                @pl.when(valid[s - NBUF])
                def _():
                    pltpu.make_async_copy(
                        buf[k], _dst(s - NBUF), sem_w[k]
                    ).wait()

            @pl.when(valid[s])
            def _():
                for j in range(CH // 16):
                    idx_v[k, pl.ds(j * 16, 16)] = jnp.minimum(
                        base_s[s] + j * 16 + lane, end_s[s] - 1
                    )
                pltpu.async_copy(
                    flat_hbm.at[idx_v.at[k]], buf[k], sem_g[k]
                )

            @pl.when(jnp.logical_not(valid[s]))
            def _():
                pltpu.async_copy(
                    zbuf, out_hbm.at[s % B, pl.ds(pos[s] * CH, ZR)], sem_z
                )
                pltpu.async_copy(
                    zbuf,
                    out_hbm.at[s % B, pl.ds(pos[s] * CH + ZR, ZR)],
                    sem_z,
                )

        sp = s - LAG
        if sp >= 0:
            kp = sp % NBUF

            @pl.when(valid[sp])
            def _():
                pltpu.make_async_copy(buf[kp], _dst(sp), sem_g[kp]).wait()
                nv = nv_s[sp]
                def _ztail(i, carry):
                    r = nv + i // 4
                    col = (i % 4) * 128
                    for u in range(8):
                        buf[kp][r, pl.ds(col + u * 16, 16)] = jnp.zeros(
                            (16,), jnp.float32
                        )
                    return carry

                lax.fori_loop(0, (CH - nv) * (D // 128), _ztail, 0)
                pltpu.async_copy(buf[kp], _dst(sp), sem_w[kp])

    for s in range(NSLOT - NBUF, NSLOT):
        @pl.when(valid[s])
        def _():
            pltpu.make_async_copy(
                buf[s % NBUF], _dst(s), sem_w[s % NBUF]
            ).wait()

    n_zero = 0
    for s in range(NSLOT):
        n_zero = n_zero + jnp.where(valid[s], 0, 2)

    def _drain_z(i, carry):
        pltpu.make_async_copy(
            zbuf, out_hbm.at[0, pl.ds(0, ZR)], sem_z
        ).wait()
        return carry

    lax.fori_loop(0, n_zero, _drain_z, 0)


def _mask_body(cu_ref, mask_ref):
    col = lax.broadcasted_iota(jnp.int32, (1, S), 1)
    for b in range(B):
        ln = cu_ref[b + 1] - cu_ref[b]
        mask_ref[pl.ds(b, 1), :] = col < ln


_mask_call = pl.pallas_call(
    _mask_body,
    in_specs=[pl.BlockSpec(memory_space=pltpu.SMEM)],
    out_specs=pl.BlockSpec(memory_space=pltpu.VMEM),
    out_shape=jax.ShapeDtypeStruct((B, S), jnp.bool_),
)


def kernel(flat, cu_seqlens):
    cu_p = jnp.pad(cu_seqlens.astype(jnp.int32), (0, 32 - (B + 1)))
    out = _sc_unflatten(flat, cu_p)
    mask = _mask_call(cu_p)
    return out, mask


# speedup vs baseline: 1.0183x; 1.0103x over previous
import functools

import jax
import jax.numpy as jnp
from jax import lax
from jax.experimental import pallas as pl
from jax.experimental.pallas import tpu as pltpu
from jax.experimental.pallas import tpu_sc as plsc

B = 16
S = 2048
T = 16384
D = 512

NW = 32
CH = 32
NPOS = S // CH
NSLOT = 32
NBUF = 6
LAG = 3

_mesh = plsc.VectorSubcoreMesh(core_axis_name="c", subcore_axis_name="s")


@functools.partial(
    pl.kernel,
    mesh=_mesh,
    out_type=jax.ShapeDtypeStruct((B, S, D), jnp.float32),
    scratch_types=[
        pltpu.VMEM((32,), jnp.int32),
        pltpu.VMEM((NBUF, CH), jnp.int32),
        pltpu.VMEM((CH, D), jnp.float32),
        pltpu.VMEM((CH, D), jnp.float32),
        pltpu.VMEM((CH, D), jnp.float32),
        pltpu.VMEM((CH, D), jnp.float32),
        pltpu.VMEM((CH, D), jnp.float32),
        pltpu.VMEM((CH, D), jnp.float32),
        pltpu.VMEM((CH, D), jnp.float32),
        pltpu.SemaphoreType.DMA,
        pltpu.SemaphoreType.DMA,
        pltpu.SemaphoreType.DMA,
        pltpu.SemaphoreType.DMA,
        pltpu.SemaphoreType.DMA,
        pltpu.SemaphoreType.DMA,
        pltpu.SemaphoreType.DMA,
        pltpu.SemaphoreType.DMA,
        pltpu.SemaphoreType.DMA,
        pltpu.SemaphoreType.DMA,
        pltpu.SemaphoreType.DMA,
        pltpu.SemaphoreType.DMA,
        pltpu.SemaphoreType.DMA,
    ],
)
def _sc_unflatten(
    flat_hbm, cu_hbm, out_hbm, cu_v, idx_v,
    buf0, buf1, buf2, buf3, buf4, buf5, zbuf,
    sg0, sg1, sg2, sg3, sg4, sg5,
    sw0, sw1, sw2, sw3, sw4, sw5, sem_z,
):
    buf = [buf0, buf1, buf2, buf3, buf4, buf5]
    sem_g = [sg0, sg1, sg2, sg3, sg4, sg5]
    sem_w = [sw0, sw1, sw2, sw3, sw4, sw5]
    wid = lax.axis_index("c") * 16 + lax.axis_index("s")

    cu_cp = pltpu.async_copy(cu_hbm, cu_v, sg0)

    def _zero(i, carry):
        r = i // 4
        col = (i % 4) * 128
        for u in range(8):
            zbuf[r, pl.ds(col + u * 16, 16)] = jnp.zeros((16,), jnp.float32)
        return carry

    lax.fori_loop(0, CH * D // 128, _zero, 0)
    cu_cp.wait()

    lane = lax.broadcasted_iota(jnp.int32, (16,), 0)

    start_e = []
    end_e = []
    for t in range(B):
        cu_b = cu_v[pl.ds(t, 16)]
        start_e.append(cu_b[0])
        end_e.append(cu_b[1])

    pos = []
    nv_s = []
    base_s = []
    end_s = []
    valid = []
    for s in range(NSLOT):
        t = s % B
        p = (wid + 2 * t + CH * (s // B)) % NPOS
        nv = jnp.clip(end_e[t] - start_e[t] - p * CH, 0, CH)
        pos.append(p)
        nv_s.append(nv)
        base_s.append(start_e[t] + p * CH)
        end_s.append(end_e[t])
        valid.append(nv > 0)

    def _dst(s):
        return out_hbm.at[s % B, pl.ds(pos[s] * CH, CH)]

    for s in range(NSLOT + LAG):
        if s < NSLOT:
            k = s % NBUF
            if s >= NBUF:
                @pl.when(valid[s - NBUF])
                def _():
                    pltpu.make_async_copy(
                        buf[k], _dst(s - NBUF), sem_w[k]
                    ).wait()

            @pl.when(valid[s])
            def _():
                for j in range(CH // 16):
                    idx_v[k, pl.ds(j * 16, 16)] = jnp.minimum(
                        base_s[s] + j * 16 + lane, end_s[s] - 1
                    )
                pltpu.async_copy(
                    flat_hbm.at[idx_v.at[k]], buf[k], sem_g[k]
                )

            @pl.when(jnp.logical_not(valid[s]))
            def _():
                pltpu.async_copy(zbuf, _dst(s), sem_z)

        sp = s - LAG
        if sp >= 0:
            kp = sp % NBUF

            @pl.when(valid[sp])
            def _():
                pltpu.make_async_copy(buf[kp], _dst(sp), sem_g[kp]).wait()
                nv = nv_s[sp]
                def _ztail(i, carry):
                    r = nv + i // 4
                    col = (i % 4) * 128
                    for u in range(8):
                        buf[kp][r, pl.ds(col + u * 16, 16)] = jnp.zeros(
                            (16,), jnp.float32
                        )
                    return carry

                lax.fori_loop(0, (CH - nv) * (D // 128), _ztail, 0)
                pltpu.async_copy(buf[kp], _dst(sp), sem_w[kp])

    for s in range(NSLOT - NBUF, NSLOT):
        @pl.when(valid[s])
        def _():
            pltpu.make_async_copy(
                buf[s % NBUF], _dst(s), sem_w[s % NBUF]
            ).wait()

    n_zero = 0
    for s in range(NSLOT):
        n_zero = n_zero + jnp.where(valid[s], 0, 1)

    def _drain_z(i, carry):
        pltpu.make_async_copy(
            zbuf, out_hbm.at[0, pl.ds(0, CH)], sem_z
        ).wait()
        return carry

    lax.fori_loop(0, n_zero, _drain_z, 0)


def _mask_body(cu_ref, mask_ref):
    col = lax.broadcasted_iota(jnp.int32, (1, S), 1)
    for b in range(B):
        ln = cu_ref[b + 1] - cu_ref[b]
        mask_ref[pl.ds(b, 1), :] = col < ln


_mask_call = pl.pallas_call(
    _mask_body,
    in_specs=[pl.BlockSpec(memory_space=pltpu.SMEM)],
    out_specs=pl.BlockSpec(memory_space=pltpu.VMEM),
    out_shape=jax.ShapeDtypeStruct((B, S), jnp.bool_),
)


def kernel(flat, cu_seqlens):
    cu_p = jnp.pad(cu_seqlens.astype(jnp.int32), (0, 32 - (B + 1)))
    out = _sc_unflatten(flat, cu_p)
    mask = _mask_call(cu_p)
    return out, mask
